# baseline (speedup 1.0000x reference)
import jax
import jax.numpy as jnp
from jax.experimental import pallas as pl
from jax.experimental.pallas import tpu as pltpu

B, H, W, C = 8, 24, 24, 384
E = 8
HW = H * W
CC = C * C
NEG = -1e30


def _moe_kernel(x_ref, wg_ref, bg_ref, be_ref, we_ref,
                out_ref, laux_ref,
                pooled_ref, xsave_ref, weflat_ref, wcomb_ref, bcomb_ref):
    s = pl.program_id(0)

    @pl.when(s < B)
    def _pool_and_save():
        xr = x_ref[...].reshape(1, HW, C)
        xsave_ref[pl.ds(s, 1)] = xr
        pool_row = jnp.mean(xr.reshape(HW, C), axis=0, keepdims=True)
        we_row = we_ref[...].astype(jnp.bfloat16).reshape(1, CC)
        for e in range(B):
            def _store(e=e):
                pooled_ref[e:e + 1, :] = pool_row
                weflat_ref[e:e + 1, :] = we_row
            pl.when(s == e)(_store)

    @pl.when(s == B)
    def _gate_and_combine():
        logits = jnp.dot(pooled_ref[...], wg_ref[...],
                         preferred_element_type=jnp.float32) + bg_ref[...]
        m = jnp.max(logits, axis=1, keepdims=True)
        eg = jnp.exp(logits - m)
        gates = eg / jnp.sum(eg, axis=1, keepdims=True)

        iota = jax.lax.broadcasted_iota(jnp.int32, (B, E), 1)
        m1 = jnp.max(gates, axis=1, keepdims=True)
        i1 = jnp.min(jnp.where(gates == m1, iota, E), axis=1, keepdims=True)
        mask1 = (iota == i1)
        g2 = jnp.where(mask1, NEG, gates)
        m2 = jnp.max(g2, axis=1, keepdims=True)
        i2 = jnp.min(jnp.where(g2 == m2, iota, E), axis=1, keepdims=True)

        e2 = jnp.exp(m2 - m1)
        denom = 1.0 + e2
        w1 = 1.0 / denom
        w2 = e2 / denom
        coeff = jnp.where(mask1, w1, 0.0) + jnp.where(iota == i2, w2, 0.0)
        bcomb_ref[...] = jnp.dot(coeff, be_ref[...],
                                 preferred_element_type=jnp.float32
                                 ).reshape(B, 1, C)
        wcomb_ref[...] = jnp.dot(coeff.astype(jnp.bfloat16), weflat_ref[...],
                                 preferred_element_type=jnp.float32
                                 ).astype(jnp.bfloat16).reshape(B, C, C)

        me = jnp.mean(gates, axis=0, keepdims=True)
        ce = jnp.mean(mask1.astype(jnp.float32), axis=0, keepdims=True)
        laux_ref[...] = jnp.sum(me * ce, axis=1, keepdims=True) * E

    @pl.when(s > B)
    def _apply():
        n = s - (B + 1)
        x_n = xsave_ref[pl.ds(n, 1)].reshape(HW, C).astype(jnp.bfloat16)
        w_n = wcomb_ref[pl.ds(n, 1)].reshape(C, C)
        y = jax.lax.dot_general(x_n, w_n, (((1,), (1,)), ((), ())),
                                preferred_element_type=jnp.float32)
        y = y + bcomb_ref[pl.ds(n, 1)].reshape(1, C)
        out_ref[...] = y.reshape(1, H, W, C)


def kernel(inputs_raw, W_gate, b_gate, W_experts, b_experts):
    bg = b_gate.reshape(1, E)

    out, laux = pl.pallas_call(
        _moe_kernel,
        grid=(2 * B + 1,),
        in_specs=[
            pl.BlockSpec((1, H, W, C),
                         lambda s: (jnp.minimum(s, B - 1), 0, 0, 0)),
            pl.BlockSpec((C, E), lambda s: (0, 0)),
            pl.BlockSpec((1, E), lambda s: (0, 0)),
            pl.BlockSpec((E, C), lambda s: (0, 0)),
            pl.BlockSpec((1, C, C),
                         lambda s: (jnp.minimum(s, E - 1), 0, 0)),
        ],
        out_specs=(
            pl.BlockSpec((1, H, W, C),
                         lambda s: (jnp.maximum(s - (B + 1), 0), 0, 0, 0)),
            pl.BlockSpec((1, 1), lambda s: (0, 0)),
        ),
        out_shape=(
            jax.ShapeDtypeStruct((B, H, W, C), jnp.float32),
            jax.ShapeDtypeStruct((1, 1), jnp.float32),
        ),
        scratch_shapes=[
            pltpu.VMEM((B, C), jnp.float32),
            pltpu.VMEM((B, HW, C), jnp.float32),
            pltpu.VMEM((E, CC), jnp.bfloat16),
            pltpu.VMEM((B, C, C), jnp.bfloat16),
            pltpu.VMEM((B, 1, C), jnp.float32),
        ],
    )(inputs_raw, W_gate, bg, b_experts, W_experts)

    return out, laux[0, 0]


# speedup vs baseline: 1.0334x; 1.0334x over previous
import jax
import jax.numpy as jnp
from jax.experimental import pallas as pl
from jax.experimental.pallas import tpu as pltpu

B, H, W, C = 8, 24, 24, 384
E = 8
HW = H * W
CC = C * C
NEG = -1e30
G = 2
NG = B // G


def _moe_kernel(x_ref, wg_ref, bg_ref, be_ref, we_ref,
                out_ref, laux_ref, weflat_ref, gates_ref):
    s = pl.program_id(0)

    @pl.when(s == 0)
    def _build_weflat():
        weflat_ref[...] = we_ref[...].astype(jnp.bfloat16).reshape(E, CC)

    xr = x_ref[...].reshape(G, HW, C)
    pooled = jnp.mean(xr, axis=1)
    logits = jnp.dot(pooled, wg_ref[...],
                     preferred_element_type=jnp.float32) + bg_ref[...]
    m = jnp.max(logits, axis=1, keepdims=True)
    eg = jnp.exp(logits - m)
    gates = eg / jnp.sum(eg, axis=1, keepdims=True)
    gates_ref[pl.ds(s, 1)] = gates.reshape(1, G, E)

    iota = jax.lax.broadcasted_iota(jnp.int32, (G, E), 1)
    m1 = jnp.max(gates, axis=1, keepdims=True)
    i1 = jnp.min(jnp.where(gates == m1, iota, E), axis=1, keepdims=True)
    mask1 = (iota == i1)
    g2 = jnp.where(mask1, NEG, gates)
    m2 = jnp.max(g2, axis=1, keepdims=True)
    i2 = jnp.min(jnp.where(g2 == m2, iota, E), axis=1, keepdims=True)

    e2 = jnp.exp(m2 - m1)
    denom = 1.0 + e2
    coeff = (jnp.where(mask1, 1.0, 0.0)
             + jnp.where(iota == i2, e2, 0.0)) / denom

    bcomb = jnp.dot(coeff, be_ref[...],
                    preferred_element_type=jnp.float32)
    wcomb = jnp.dot(coeff.astype(jnp.bfloat16), weflat_ref[...],
                    preferred_element_type=jnp.float32
                    ).astype(jnp.bfloat16).reshape(G, C, C)

    for i in range(G):
        y = jax.lax.dot_general(
            xr[i].astype(jnp.bfloat16), wcomb[i],
            (((1,), (1,)), ((), ())),
            preferred_element_type=jnp.float32)
        y = y + bcomb[i:i + 1]
        out_ref[i:i + 1] = y.reshape(1, H, W, C)

    @pl.when(s == NG - 1)
    def _aux_loss():
        gall = gates_ref[...].reshape(B, E)
        iota_b = jax.lax.broadcasted_iota(jnp.int32, (B, E), 1)
        mb1 = jnp.max(gall, axis=1, keepdims=True)
        ib1 = jnp.min(jnp.where(gall == mb1, iota_b, E),
                      axis=1, keepdims=True)
        maskb = (iota_b == ib1).astype(jnp.float32)
        me = jnp.mean(gall, axis=0, keepdims=True)
        ce = jnp.mean(maskb, axis=0, keepdims=True)
        laux_ref[...] = jnp.sum(me * ce, axis=1, keepdims=True) * E

def kernel(inputs_raw, W_gate, b_gate, W_experts, b_experts):
    bg = b_gate.reshape(1, E)

    out, laux = pl.pallas_call(
        _moe_kernel,
        grid=(NG,),
        in_specs=[
            pl.BlockSpec((G, H, W, C), lambda s: (s, 0, 0, 0)),
            pl.BlockSpec((C, E), lambda s: (0, 0)),
            pl.BlockSpec((1, E), lambda s: (0, 0)),
            pl.BlockSpec((E, C), lambda s: (0, 0)),
            pl.BlockSpec((E, C, C), lambda s: (0, 0, 0)),
        ],
        out_specs=(
            pl.BlockSpec((G, H, W, C), lambda s: (s, 0, 0, 0)),
            pl.BlockSpec((1, 1), lambda s: (0, 0)),
        ),
        out_shape=(
            jax.ShapeDtypeStruct((B, H, W, C), jnp.float32),
            jax.ShapeDtypeStruct((1, 1), jnp.float32),
        ),
        scratch_shapes=[
            pltpu.VMEM((E, CC), jnp.bfloat16),
            pltpu.VMEM((NG, G, E), jnp.float32),
        ],
    )(inputs_raw, W_gate, bg, b_experts, W_experts)

    return out, laux[0, 0]


# speedup vs baseline: 1.3094x; 1.2671x over previous
import jax
import jax.numpy as jnp
from jax.experimental import pallas as pl
from jax.experimental.pallas import tpu as pltpu

B, H, W, C = 8, 24, 24, 384
E = 8
HW = H * W
CC = C * C
NEG = -1e30


def _moe_kernel(x_ref, wg_ref, bg_ref, be_ref, we_hbm,
                out_ref, laux_ref,
                wescr_ref, bcomb_ref, wcomb_ref, sem):
    n = pl.program_id(0)

    @pl.when(n == 0)
    def _gate_and_combine():
        copy = pltpu.make_async_copy(we_hbm, wescr_ref, sem)
        copy.start()

        xall = x_ref[...].reshape(B * HW, C)
        r = jax.lax.broadcasted_iota(jnp.int32, (B, B * HW), 0)
        c = jax.lax.broadcasted_iota(jnp.int32, (B, B * HW), 1)
        avg = jnp.where((c >= r * HW) & (c < (r + 1) * HW),
                        jnp.float32(1.0 / HW), jnp.float32(0.0))
        pooled = jnp.dot(avg, xall, preferred_element_type=jnp.float32)

        logits = jnp.dot(pooled, wg_ref[...],
                         preferred_element_type=jnp.float32) + bg_ref[...]
        m = jnp.max(logits, axis=1, keepdims=True)
        eg = jnp.exp(logits - m)
        gates = eg / jnp.sum(eg, axis=1, keepdims=True)

        iota = jax.lax.broadcasted_iota(jnp.int32, (B, E), 1)
        m1 = jnp.max(gates, axis=1, keepdims=True)
        i1 = jnp.min(jnp.where(gates == m1, iota, E), axis=1, keepdims=True)
        mask1 = (iota == i1)
        g2 = jnp.where(mask1, NEG, gates)
        m2 = jnp.max(g2, axis=1, keepdims=True)
        i2 = jnp.min(jnp.where(g2 == m2, iota, E), axis=1, keepdims=True)

        e2 = jnp.exp(m2 - m1)
        coeff = (jnp.where(mask1, 1.0, 0.0)
                 + jnp.where(iota == i2, e2, 0.0)) / (1.0 + e2)
        bcomb_ref[...] = jnp.dot(coeff, be_ref[...],
                                 preferred_element_type=jnp.float32)

        me = jnp.mean(gates, axis=0, keepdims=True)
        ce = jnp.mean(mask1.astype(jnp.float32), axis=0, keepdims=True)
        laux_ref[...] = jnp.sum(me * ce, axis=1, keepdims=True) * E

        copy.wait()
        we_flat = wescr_ref[...].astype(jnp.bfloat16).reshape(E, CC)
        wcomb_ref[...] = jnp.dot(coeff.astype(jnp.bfloat16), we_flat,
                                 preferred_element_type=jnp.float32
                                 ).astype(jnp.bfloat16).reshape(B, C, C)

    x_n = x_ref[pl.ds(n, 1)].reshape(HW, C).astype(jnp.bfloat16)
    w_n = wcomb_ref[pl.ds(n, 1)].reshape(C, C)
    y = jax.lax.dot_general(x_n, w_n, (((1,), (1,)), ((), ())),
                            preferred_element_type=jnp.float32)
    y = y + bcomb_ref[pl.ds(n, 1)]
    out_ref[...] = y.reshape(1, H, W, C)


def kernel(inputs_raw, W_gate, b_gate, W_experts, b_experts):
    bg = b_gate.reshape(1, E)

    out, laux = pl.pallas_call(
        _moe_kernel,
        grid=(B,),
        in_specs=[
            pl.BlockSpec((B, H, W, C), lambda n: (0, 0, 0, 0)),
            pl.BlockSpec((C, E), lambda n: (0, 0)),
            pl.BlockSpec((1, E), lambda n: (0, 0)),
            pl.BlockSpec((E, C), lambda n: (0, 0)),
            pl.BlockSpec(memory_space=pltpu.MemorySpace.HBM),
        ],
        out_specs=(
            pl.BlockSpec((1, H, W, C), lambda n: (n, 0, 0, 0)),
            pl.BlockSpec((1, 1), lambda n: (0, 0)),
        ),
        out_shape=(
            jax.ShapeDtypeStruct((B, H, W, C), jnp.float32),
            jax.ShapeDtypeStruct((1, 1), jnp.float32),
        ),
        scratch_shapes=[
            pltpu.VMEM((E, C, C), jnp.float32),
            pltpu.VMEM((B, C), jnp.float32),
            pltpu.VMEM((B, C, C), jnp.bfloat16),
            pltpu.SemaphoreType.DMA,
        ],
    )(inputs_raw, W_gate, bg, b_experts, W_experts)

    return out, laux[0, 0]


# speedup vs baseline: 1.4477x; 1.1056x over previous
import jax
import jax.numpy as jnp
from jax.experimental import pallas as pl
from jax.experimental.pallas import tpu as pltpu

B, H, W, C = 8, 24, 24, 384
E = 8
HW = H * W
CC = C * C
NEG = -1e30
GRP = 2


def _moe_kernel(x_ref, wg_ref, bg_ref, be_ref, we_hbm,
                out_ref, laux_ref,
                wescr_ref, bcomb_ref, wcomb_ref, sem):
    n = pl.program_id(0)

    @pl.when(n == 0)
    def _gate_and_combine():
        copy = pltpu.make_async_copy(we_hbm, wescr_ref, sem)
        copy.start()

        xall = x_ref[...].reshape(B * HW, C)
        r = jax.lax.broadcasted_iota(jnp.int32, (B, B * HW), 0)
        c = jax.lax.broadcasted_iota(jnp.int32, (B, B * HW), 1)
        avg = jnp.where((c >= r * HW) & (c < (r + 1) * HW),
                        jnp.float32(1.0 / HW), jnp.float32(0.0))
        pooled = jnp.dot(avg, xall, preferred_element_type=jnp.float32)

        logits = jnp.dot(pooled, wg_ref[...],
                         preferred_element_type=jnp.float32) + bg_ref[...]
        m = jnp.max(logits, axis=1, keepdims=True)
        eg = jnp.exp(logits - m)
        gates = eg / jnp.sum(eg, axis=1, keepdims=True)

        iota = jax.lax.broadcasted_iota(jnp.int32, (B, E), 1)
        m1 = jnp.max(gates, axis=1, keepdims=True)
        i1 = jnp.min(jnp.where(gates == m1, iota, E), axis=1, keepdims=True)
        mask1 = (iota == i1)
        g2 = jnp.where(mask1, NEG, gates)
        m2 = jnp.max(g2, axis=1, keepdims=True)
        i2 = jnp.min(jnp.where(g2 == m2, iota, E), axis=1, keepdims=True)

        e2 = jnp.exp(m2 - m1)
        coeff = (jnp.where(mask1, 1.0, 0.0)
                 + jnp.where(iota == i2, e2, 0.0)) / (1.0 + e2)
        bcomb_ref[...] = jnp.dot(coeff, be_ref[...],
                                 preferred_element_type=jnp.float32)

        me = jnp.mean(gates, axis=0, keepdims=True)
        ce = jnp.mean(mask1.astype(jnp.float32), axis=0, keepdims=True)
        laux_ref[...] = jnp.sum(me * ce, axis=1, keepdims=True) * E

        copy.wait()
        we_flat = wescr_ref[...].astype(jnp.bfloat16).reshape(E, CC)
        wcomb_ref[...] = jnp.dot(coeff.astype(jnp.bfloat16), we_flat,
                                 preferred_element_type=jnp.float32
                                 ).astype(jnp.bfloat16).reshape(B, C, C)

    for i in range(GRP):
        img = n * GRP + i
        x_n = x_ref[pl.ds(img, 1)].reshape(HW, C).astype(jnp.bfloat16)
        w_n = wcomb_ref[pl.ds(img, 1)].reshape(C, C)
        y = jax.lax.dot_general(x_n, w_n, (((1,), (1,)), ((), ())),
                                preferred_element_type=jnp.float32)
        y = y + bcomb_ref[pl.ds(img, 1)]
        out_ref[i:i + 1] = y.reshape(1, H, W, C)


def kernel(inputs_raw, W_gate, b_gate, W_experts, b_experts):
    bg = b_gate.reshape(1, E)

    out, laux = pl.pallas_call(
        _moe_kernel,
        grid=(B // GRP,),
        in_specs=[
            pl.BlockSpec((B, H, W, C), lambda n: (0, 0, 0, 0)),
            pl.BlockSpec((C, E), lambda n: (0, 0)),
            pl.BlockSpec((1, E), lambda n: (0, 0)),
            pl.BlockSpec((E, C), lambda n: (0, 0)),
            pl.BlockSpec(memory_space=pltpu.MemorySpace.HBM),
        ],
        out_specs=(
            pl.BlockSpec((GRP, H, W, C), lambda n: (n, 0, 0, 0)),
            pl.BlockSpec((1, 1), lambda n: (0, 0)),
        ),
        out_shape=(
            jax.ShapeDtypeStruct((B, H, W, C), jnp.float32),
            jax.ShapeDtypeStruct((1, 1), jnp.float32),
        ),
        scratch_shapes=[
            pltpu.VMEM((E, C, C), jnp.float32),
            pltpu.VMEM((B, C), jnp.float32),
            pltpu.VMEM((B, C, C), jnp.bfloat16),
            pltpu.SemaphoreType.DMA,
        ],
    )(inputs_raw, W_gate, bg, b_experts, W_experts)

    return out, laux[0, 0]


# speedup vs baseline: 1.4685x; 1.0144x over previous
import jax
import jax.numpy as jnp
from jax.experimental import pallas as pl
from jax.experimental.pallas import tpu as pltpu

B, H, W, C = 8, 24, 24, 384
E = 8
HW = H * W
CC = C * C
NEG = -1e30
GRP = 4


def _moe_kernel(x_ref, wg_ref, bg_ref, be_ref, we_hbm,
                out_ref, laux_ref,
                wescr_ref, bcomb_ref, wcomb_ref, sem):
    n = pl.program_id(0)

    @pl.when(n == 0)
    def _gate_and_combine():
        copy = pltpu.make_async_copy(we_hbm, wescr_ref, sem)
        copy.start()

        xall = x_ref[...].reshape(B * HW, C)
        r = jax.lax.broadcasted_iota(jnp.int32, (B, B * HW), 0)
        c = jax.lax.broadcasted_iota(jnp.int32, (B, B * HW), 1)
        avg = jnp.where((c >= r * HW) & (c < (r + 1) * HW),
                        jnp.float32(1.0 / HW), jnp.float32(0.0))
        pooled = jnp.dot(avg, xall, preferred_element_type=jnp.float32)

        logits = jnp.dot(pooled, wg_ref[...],
                         preferred_element_type=jnp.float32) + bg_ref[...]
        m = jnp.max(logits, axis=1, keepdims=True)
        eg = jnp.exp(logits - m)
        gates = eg / jnp.sum(eg, axis=1, keepdims=True)

        iota = jax.lax.broadcasted_iota(jnp.int32, (B, E), 1)
        m1 = jnp.max(gates, axis=1, keepdims=True)
        i1 = jnp.min(jnp.where(gates == m1, iota, E), axis=1, keepdims=True)
        mask1 = (iota == i1)
        g2 = jnp.where(mask1, NEG, gates)
        m2 = jnp.max(g2, axis=1, keepdims=True)
        i2 = jnp.min(jnp.where(g2 == m2, iota, E), axis=1, keepdims=True)

        e2 = jnp.exp(m2 - m1)
        coeff = (jnp.where(mask1, 1.0, 0.0)
                 + jnp.where(iota == i2, e2, 0.0)) / (1.0 + e2)
        bcomb_ref[...] = jnp.dot(coeff, be_ref[...],
                                 preferred_element_type=jnp.float32)

        me = jnp.mean(gates, axis=0, keepdims=True)
        ce = jnp.mean(mask1.astype(jnp.float32), axis=0, keepdims=True)
        laux_ref[...] = jnp.sum(me * ce, axis=1, keepdims=True) * E

        copy.wait()
        we_flat = wescr_ref[...].astype(jnp.bfloat16).reshape(E, CC)
        wcomb_ref[...] = jnp.dot(coeff.astype(jnp.bfloat16), we_flat,
                                 preferred_element_type=jnp.float32
                                 ).astype(jnp.bfloat16).reshape(B, C, C)

    for i in range(GRP):
        img = n * GRP + i
        x_n = x_ref[pl.ds(img, 1)].reshape(HW, C).astype(jnp.bfloat16)
        w_n = wcomb_ref[pl.ds(img, 1)].reshape(C, C)
        y = jax.lax.dot_general(x_n, w_n, (((1,), (1,)), ((), ())),
                                preferred_element_type=jnp.float32)
        y = y + bcomb_ref[pl.ds(img, 1)]
        out_ref[i:i + 1] = y.reshape(1, H, W, C)


def kernel(inputs_raw, W_gate, b_gate, W_experts, b_experts):
    bg = b_gate.reshape(1, E)

    out, laux = pl.pallas_call(
        _moe_kernel,
        grid=(B // GRP,),
        in_specs=[
            pl.BlockSpec((B, H, W, C), lambda n: (0, 0, 0, 0)),
            pl.BlockSpec((C, E), lambda n: (0, 0)),
            pl.BlockSpec((1, E), lambda n: (0, 0)),
            pl.BlockSpec((E, C), lambda n: (0, 0)),
            pl.BlockSpec(memory_space=pltpu.MemorySpace.HBM),
        ],
        out_specs=(
            pl.BlockSpec((GRP, H, W, C), lambda n: (n, 0, 0, 0)),
            pl.BlockSpec((1, 1), lambda n: (0, 0)),
        ),
        out_shape=(
            jax.ShapeDtypeStruct((B, H, W, C), jnp.float32),
            jax.ShapeDtypeStruct((1, 1), jnp.float32),
        ),
        scratch_shapes=[
            pltpu.VMEM((E, C, C), jnp.float32),
            pltpu.VMEM((B, C), jnp.float32),
            pltpu.VMEM((B, C, C), jnp.bfloat16),
            pltpu.SemaphoreType.DMA,
        ],
    )(inputs_raw, W_gate, bg, b_experts, W_experts)

    return out, laux[0, 0]
